# baseline (speedup 1.0000x reference)
import functools
import jax
import jax.numpy as jnp
from jax import lax
from jax.experimental import pallas as pl
from jax.experimental.pallas import tpu as pltpu
from jax.experimental.pallas import tpu_sc as plsc

N = 10000
E = 320000
H = 128
MIN_DISP_SCALE = 1e-06
LOG_MULT_BOUND = 4.0
CLAMP_LO, CLAMP_HI = 0.0, 30.0
YIELD = 2.5e8

_NC = 2
_NS = 16
_NW = _NC * _NS
_L = 16
_C = 80
_EW = E // _NW
_KW = _EW // _C
_NPAD = 10240
_ZR = 128
_RPT = _NPAD // _NS


_NBUF = 3
_EA_SCALE = 2048.0


def _agg_body(h_hbm, src_hbm, dst_hbm, ea_hbm, out_hbm,
              agg_sh, sidx, didx, rows_v, eab_v, sisem, disem, esem, gsem,
              ssem):
    c = lax.axis_index("c")
    s = lax.axis_index("s")
    wid = s * _NC + c

    zero16 = jnp.zeros((_L,), jnp.float32)

    def zrow(r, carry):
        for cc in range(H // _L):
            rows_v[0, r, pl.ds(cc * _L, _L)] = zero16
        return carry

    lax.fori_loop(0, _C, zrow, 0)
    base_rows = s * _RPT
    for j in range(_RPT // _C):
        pltpu.sync_copy(rows_v.at[0], agg_sh.at[pl.ds(base_rows + j * _C, _C)])
    plsc.subcore_barrier()

    ebase = wid * _EW

    def idx_issue(g, b):
        off = ebase + g * _C
        pltpu.async_copy(src_hbm.at[pl.ds(off, _C)], sidx.at[b], sisem.at[b])
        pltpu.async_copy(dst_hbm.at[pl.ds(off, _C)], didx.at[b], disem.at[b])

    def sidx_wait(b):
        pltpu.make_async_copy(src_hbm.at[pl.ds(0, _C)], sidx.at[b],
                              sisem.at[b]).wait()

    def didx_wait(b):
        pltpu.make_async_copy(dst_hbm.at[pl.ds(0, _C)], didx.at[b],
                              disem.at[b]).wait()

    def ea_issue(g, b):
        off2 = pl.multiple_of(wid * (_EW // 2) + g * (_C // 2), 8)
        pltpu.async_copy(ea_hbm.at[pl.ds(off2, _C // 2)],
                         eab_v.at[b], esem.at[b])

    def ea_wait(b):
        pltpu.make_async_copy(ea_hbm.at[pl.ds(0, _C // 2)], eab_v.at[b],
                              esem.at[b]).wait()

    def gather_issue(b):
        pltpu.async_copy(h_hbm.at[sidx.at[b]], rows_v.at[b], gsem.at[b])

    def gather_wait(b):
        pltpu.make_async_copy(h_hbm.at[pl.ds(0, _C)], rows_v.at[b],
                              gsem.at[b]).wait()

    def scatter_issue(b):
        pltpu.async_copy(rows_v.at[b], agg_sh.at[didx.at[b]], ssem.at[b],
                         add=True)

    def scatter_wait(b):
        pltpu.make_async_copy(rows_v.at[b], agg_sh.at[pl.ds(0, _C)],
                              ssem.at[b]).wait()

    def stage_a(g, b, first=False):
        if not first:
            scatter_wait(b)
        idx_issue(g, b)
        ea_issue(g, b)

    def stage_b(b):
        sidx_wait(b)
        gather_issue(b)

    def compute_scatter(b):
        gather_wait(b)
        ea_wait(b)

        shift16 = jnp.full((_L,), 16, jnp.int32)
        inv_scale = jnp.full((_L,), 1.0 / _EA_SCALE, jnp.float32)

        def crow(rr, carry):
            for half in range(2):
                r = 2 * rr + half
                for q in range(H // (2 * _L)):
                    w = eab_v[b, rr, pl.ds(H // 2 * half + _L * q, _L)]
                    ve = ((w << shift16) >> shift16).astype(jnp.float32) * inv_scale
                    vo = (w >> shift16).astype(jnp.float32) * inv_scale
                    sl0 = pl.ds(_L * q, _L)
                    sl1 = pl.ds(H // 2 + _L * q, _L)
                    rows_v[b, r, sl0] = jnp.maximum(rows_v[b, r, sl0] + ve, 0.0)
                    rows_v[b, r, sl1] = jnp.maximum(rows_v[b, r, sl1] + vo, 0.0)
            return carry

        lax.fori_loop(0, _C // 2, crow, 0)
        didx_wait(b)
        scatter_issue(b)

    stage_a(0, 0, first=True)
    stage_a(1, 1, first=True)
    stage_b(0)
    stage_a(2, 2, first=True)
    stage_b(1)
    compute_scatter(0)

    def group(gg, carry):
        for j in range(_NBUF):
            g = 1 + gg * _NBUF + j
            b = (1 + j) % _NBUF
            stage_a(g + 2, (b + 2) % _NBUF)
            stage_b((b + 1) % _NBUF)
            compute_scatter(b)
        return carry

    lax.fori_loop(0, (_KW - 5) // _NBUF, group, 0)

    stage_a(_KW - 2, 0)
    stage_b(2)
    compute_scatter(1)
    stage_a(_KW - 1, 1)
    stage_b(0)
    compute_scatter(2)
    stage_b(1)
    compute_scatter(0)
    compute_scatter(1)
    for b in range(_NBUF):
        scatter_wait(b)

    plsc.subcore_barrier()

    for j in range(_RPT // _C):
        r0 = base_rows + j * _C
        pltpu.sync_copy(agg_sh.at[pl.ds(r0, _C)], out_hbm.at[c, pl.ds(r0, _C)])


_sc_aggregate = pl.kernel(
    _agg_body,
    out_type=jax.ShapeDtypeStruct((_NC, _NPAD, H), jnp.float32),
    compiler_params=pltpu.CompilerParams(needs_layout_passes=False),
    mesh=plsc.VectorSubcoreMesh(core_axis_name="c", subcore_axis_name="s"),
    scratch_types=[
        pltpu.VMEM_SHARED((_NPAD, H), jnp.float32),
        pltpu.VMEM((_NBUF, _C), jnp.int32),
        pltpu.VMEM((_NBUF, _C), jnp.int32),
        pltpu.VMEM((_NBUF, _C, H), jnp.float32),
        pltpu.VMEM((_NBUF, _C // 2, H), jnp.int32),
        pltpu.SemaphoreType.DMA((_NBUF,)),
        pltpu.SemaphoreType.DMA((_NBUF,)),
        pltpu.SemaphoreType.DMA((_NBUF,)),
        pltpu.SemaphoreType.DMA((_NBUF,)),
        pltpu.SemaphoreType.DMA((_NBUF,)),
    ],
)


def _ln_rows(y, g, bt):
    mu = jnp.mean(y, axis=-1, keepdims=True)
    var = jnp.mean((y - mu) ** 2, axis=-1, keepdims=True)
    return (y - mu) / jnp.sqrt(var + 1e-5) * g + bt


def _mm(a, b):
    return jnp.dot(a, b, preferred_element_type=jnp.float32)


def _enc_body(x_ref, w1, b1, w2, b2, g, bt, o_ref):
    t = jnp.maximum(_mm(x_ref[...], w1[...]) + b1[...], 0.0)
    y = _mm(t, w2[...]) + b2[...]
    o_ref[...] = _ln_rows(y, g[...], bt[...])


def _full(ref_shape):
    return pl.BlockSpec(ref_shape, lambda i: (0, 0))


def _edge_enc_body(x_ref, w1, b1, w2, b2, g, bt, o_ref):
    t = jnp.maximum(_mm(x_ref[...], w1[...]) + b1[...], 0.0)
    y = _mm(t, w2[...]) + b2[...]
    z = _ln_rows(y, g[...], bt[...])
    q = jnp.clip(jnp.round(z * _EA_SCALE), -32768.0, 32767.0).astype(jnp.int32)
    lo = q[:, :H // 2] & 65535
    hi = q[:, H // 2:] << 16
    o_ref[...] = hi | lo


def _edge_encode(inp, w1, b1, w2, b2, g, bt, rb):
    n, d = inp.shape
    grid = n // rb
    return pl.pallas_call(
        _edge_enc_body,
        grid=(grid,),
        in_specs=[
            pl.BlockSpec((rb, d), lambda i: (i, 0)),
            _full(w1.shape), _full(b1.shape), _full(w2.shape),
            _full(b2.shape), _full(g.shape), _full(bt.shape),
        ],
        out_specs=pl.BlockSpec((rb, H // 2), lambda i: (i, 0)),
        out_shape=jax.ShapeDtypeStruct((n, H // 2), jnp.int32),
    )(inp, w1, b1, w2, b2, g, bt)


def _encode(inp, w1, b1, w2, b2, g, bt, rb):
    n, d = inp.shape
    grid = n // rb
    return pl.pallas_call(
        _enc_body,
        grid=(grid,),
        in_specs=[
            pl.BlockSpec((rb, d), lambda i: (i, 0)),
            _full(w1.shape), _full(b1.shape), _full(w2.shape),
            _full(b2.shape), _full(g.shape), _full(bt.shape),
        ],
        out_specs=pl.BlockSpec((rb, H), lambda i: (i, 0)),
        out_shape=jax.ShapeDtypeStruct((n, H), jnp.float32),
    )(inp, w1, b1, w2, b2, g, bt)


def _upd_body(h_ref, a0_ref, a1_ref, A1, a1b, A2, a2b, g, bt, o_ref):
    h = h_ref[...]
    z = h + a0_ref[0] + a1_ref[0]
    t = jnp.maximum(_mm(z, A1[...]) + a1b[...], 0.0)
    hh = _mm(t, A2[...]) + a2b[...]
    o_ref[...] = _ln_rows(h + jnp.maximum(hh, 0.0), g[...], bt[...])


def _layer_update(h, parts, A1, a1b, A2, a2b, g, bt, rb=2000):
    grid = N // rb
    blk = pl.BlockSpec((rb, H), lambda i: (i, 0))
    p0 = pl.BlockSpec((1, rb, H), lambda i: (0, i, 0))
    p1 = pl.BlockSpec((1, rb, H), lambda i: (1, i, 0))
    return pl.pallas_call(
        _upd_body,
        grid=(grid,),
        in_specs=[blk, p0, p1, _full(A1.shape), _full(a1b.shape),
                  _full(A2.shape), _full(a2b.shape), _full(g.shape),
                  _full(bt.shape)],
        out_specs=blk,
        out_shape=jax.ShapeDtypeStruct((N, H), jnp.float32),
    )(h, parts, parts, A1, a1b, A2, a2b, g, bt)


def _heads_body(h_ref, dW1, db1, dW2, db2, sW1, sb1, sW2, sb2,
                feats, spW1, spb1, spW2, spb2, spg, spbt, spW3, spb3, lb,
                rawu_ref, u_ref, logs_ref, s_ref, safety_ref, disp_ref):
    h = h_ref[...]
    t = jnp.maximum(_mm(h, dW1[...]) + db1[...], 0.0)
    raw = _mm(t, dW2[...]) + db2[...]
    rms = jnp.maximum(jnp.sqrt(jnp.sum(raw * raw) / N), 1e-8)

    t2 = jnp.maximum(_mm(h, sW1[...]) + sb1[...], 0.0)
    ls = jnp.clip(_mm(t2, sW2[...]) + sb2[...], CLAMP_LO, CLAMP_HI)
    logs_ref[...] = ls
    sv = jnp.exp(ls)
    s_ref[...] = sv
    safety_ref[...] = YIELD / (sv + 1e-8)

    hs = jnp.maximum(_mm(feats[...], spW1[...]) + spb1[...], 0.0)
    hs = _mm(hs, spW2[...]) + spb2[...]
    hs = jnp.maximum(_ln_rows(hs, spg[...], spbt[...]), 0.0)
    lm = _mm(hs, spW3[...]) + spb3[...]
    lm = LOG_MULT_BOUND * jnp.tanh(lm / LOG_MULT_BOUND)
    base = MIN_DISP_SCALE + jnp.log1p(jnp.exp(lb[0, 0]))
    disp = jnp.maximum(base * jnp.exp(lm), MIN_DISP_SCALE)
    disp_ref[...] = disp

    ru = raw / rms
    rawu_ref[...] = ru
    u_ref[...] = ru * disp[0, 0]


def _heads(h, dW1, db1, dW2, db2, sW1, sb1, sW2, sb2,
           feats, spW1, spb1, spW2, spb2, spg, spbt, spW3, spb3, lb):
    return pl.pallas_call(
        _heads_body,
        out_shape=[
            jax.ShapeDtypeStruct((N, 3), jnp.float32),
            jax.ShapeDtypeStruct((N, 3), jnp.float32),
            jax.ShapeDtypeStruct((N, 1), jnp.float32),
            jax.ShapeDtypeStruct((N, 1), jnp.float32),
            jax.ShapeDtypeStruct((N, 1), jnp.float32),
            jax.ShapeDtypeStruct((1, 1), jnp.float32),
        ],
    )(h, dW1, db1, dW2, db2, sW1, sb1, sW2, sb2,
      feats, spW1, spb1, spW2, spb2, spg, spbt, spW3, spb3, lb)


def _row(v):
    return v.reshape(1, -1)


def kernel(x, edge_index, edge_attr, params):
    src = jnp.asarray(edge_index[0], jnp.int32)
    dst = jnp.asarray(edge_index[1], jnp.int32)

    pe = params['node_enc']
    h = _encode(x, pe['W1'], _row(pe['b1']), pe['W2'], _row(pe['b2']),
                _row(pe['g']), _row(pe['bt']), rb=2000)
    pg = params['edge_enc']
    ea = _edge_encode(edge_attr, pg['W1'], _row(pg['b1']),
                      pg['W2'], _row(pg['b2']),
                      _row(pg['g']), _row(pg['bt']), rb=2560)
    ea = ea.reshape(E // 2, H)

    for cp in params['convs']:
        parts = _sc_aggregate(h, src, dst, ea)
        h = _layer_update(h, parts, cp['A1'], _row(cp['a1']), cp['A2'],
                          _row(cp['a2']), _row(cp['g']), _row(cp['bt']))

    one = jnp.ones((1, 1), dtype=jnp.float32)
    logF = jnp.log(one + 1.0)
    logE = jnp.log(jnp.full((1, 1), 2.1e11, dtype=jnp.float32) + 1e-12)
    nu = jnp.full((1, 1), 0.3, dtype=jnp.float32)
    logL = jnp.log(one + 1e-6)
    logI = jnp.log(one + 1e-18)
    phys = logF + 3.0 * logL - logE - logI
    feats = jnp.concatenate([logF, logE, nu, logL, logI, phys], axis=-1)

    dp = params['disp_head']
    st = params['stress_head']
    sp = params['scale_mlp']
    raw_u, u, log_s, s, safety, disp = _heads(
        h, dp['W1'], _row(dp['b1']), dp['W2'], _row(dp['b2']),
        st['W1'], _row(st['b1']), st['W2'], _row(st['b2']),
        feats, sp['W1'], _row(sp['b1']), sp['W2'], _row(sp['b2']),
        _row(sp['g']), _row(sp['bt']), sp['W3'], _row(sp['b3']),
        params['log_base'].reshape(1, 1))

    return (u, raw_u, s, log_s, disp[0, 0], disp, safety)


# speedup vs baseline: 2.0307x; 2.0307x over previous
import functools
import jax
import jax.numpy as jnp
from jax import lax
from jax.experimental import pallas as pl
from jax.experimental.pallas import tpu as pltpu
from jax.experimental.pallas import tpu_sc as plsc

N = 10000
E = 320000
H = 128
MIN_DISP_SCALE = 1e-06
LOG_MULT_BOUND = 4.0
CLAMP_LO, CLAMP_HI = 0.0, 30.0
YIELD = 2.5e8

_NC = 2
_NS = 16
_NW = _NC * _NS
_L = 16
_C = 80
_EW = E // _NW
_KW = _EW // _C
_NPAD = 10240
_ZR = 128
_RPT = _NPAD // _NS


_NBUF = 4


def _agg_body(h_hbm, src_hbm, dst_hbm, ea_hbm, out_hbm,
              agg_sh, sidx, didx, rows_v, sisem, disem, esem, gsem, ssem):
    c = lax.axis_index("c")
    s = lax.axis_index("s")
    wid = s * _NC + c

    zero16 = jnp.zeros((_L,), jnp.float32)

    def zrow(r, carry):
        for cc in range(H // _L):
            rows_v[0, r, pl.ds(cc * _L, _L)] = zero16
        return carry

    lax.fori_loop(0, _C, zrow, 0)
    base_rows = s * _RPT
    for j in range(_RPT // _C):
        pltpu.sync_copy(rows_v.at[0], agg_sh.at[pl.ds(base_rows + j * _C, _C)])
    plsc.subcore_barrier()

    ebase = wid * _EW

    def idx_issue(g, b):
        off = ebase + g * _C
        pltpu.async_copy(src_hbm.at[pl.ds(off, _C)], sidx.at[b], sisem.at[b])
        pltpu.async_copy(dst_hbm.at[pl.ds(off, _C)], didx.at[b], disem.at[b])

    def sidx_wait(b):
        pltpu.make_async_copy(src_hbm.at[pl.ds(0, _C)], sidx.at[b],
                              sisem.at[b]).wait()

    def didx_wait(b):
        pltpu.make_async_copy(dst_hbm.at[pl.ds(0, _C)], didx.at[b],
                              disem.at[b]).wait()

    def ea_issue(g, b):
        pltpu.async_copy(ea_hbm.at[pl.ds(ebase + g * _C, _C)],
                         rows_v.at[b], esem.at[b])

    def ea_wait(b):
        pltpu.make_async_copy(ea_hbm.at[pl.ds(0, _C)], rows_v.at[b],
                              esem.at[b]).wait()

    def gather_issue(b):
        pltpu.async_copy(h_hbm.at[sidx.at[b]], rows_v.at[b], gsem.at[b],
                         add=True)

    def gather_wait(b):
        pltpu.make_async_copy(h_hbm.at[pl.ds(0, _C)], rows_v.at[b],
                              gsem.at[b]).wait()

    def scatter_issue(b):
        pltpu.async_copy(rows_v.at[b], agg_sh.at[didx.at[b]], ssem.at[b],
                         add=True)

    def scatter_wait(b):
        pltpu.make_async_copy(rows_v.at[b], agg_sh.at[pl.ds(0, _C)],
                              ssem.at[b]).wait()

    def stage_a(g, b, first=False):
        if not first:
            scatter_wait(b)
        idx_issue(g, b)
        ea_issue(g, b)

    def stage_b(b):
        sidx_wait(b)
        ea_wait(b)
        gather_issue(b)

    def compute_scatter(b):
        gather_wait(b)

        def crow(r, carry):
            for cc in range(H // _L):
                sl = pl.ds(cc * _L, _L)
                rows_v[b, r, sl] = jnp.maximum(rows_v[b, r, sl], 0.0)
            return carry

        lax.fori_loop(0, _C, crow, 0)
        didx_wait(b)
        scatter_issue(b)

    stage_a(0, 0, first=True)
    stage_a(1, 1, first=True)
    stage_a(2, 2, first=True)
    stage_b(0)
    stage_b(1)
    stage_a(3, 3, first=True)
    stage_b(2)
    compute_scatter(0)

    def group(gg, carry):
        for j in range(_NBUF):
            g = 1 + gg * _NBUF + j
            b = (1 + j) % _NBUF
            stage_a(g + 3, (b + 3) % _NBUF)
            stage_b((b + 2) % _NBUF)
            compute_scatter(b)
        return carry

    lax.fori_loop(0, (_KW - 5) // _NBUF, group, 0)

    stage_a(_KW - 1, 0)
    stage_b(3)
    compute_scatter(1)
    stage_b(0)
    compute_scatter(2)
    compute_scatter(3)
    compute_scatter(0)
    for b in range(_NBUF):
        scatter_wait(b)

    plsc.subcore_barrier()

    for j in range(_RPT // _C):
        r0 = base_rows + j * _C
        pltpu.sync_copy(agg_sh.at[pl.ds(r0, _C)], out_hbm.at[c, pl.ds(r0, _C)])


_sc_aggregate = pl.kernel(
    _agg_body,
    out_type=jax.ShapeDtypeStruct((_NC, _NPAD, H), jnp.float32),
    mesh=plsc.VectorSubcoreMesh(core_axis_name="c", subcore_axis_name="s"),
    scratch_types=[
        pltpu.VMEM_SHARED((_NPAD, H), jnp.float32),
        pltpu.VMEM((_NBUF, _C), jnp.int32),
        pltpu.VMEM((_NBUF, _C), jnp.int32),
        pltpu.VMEM((_NBUF, _C, H), jnp.float32),
        pltpu.SemaphoreType.DMA((_NBUF,)),
        pltpu.SemaphoreType.DMA((_NBUF,)),
        pltpu.SemaphoreType.DMA((_NBUF,)),
        pltpu.SemaphoreType.DMA((_NBUF,)),
        pltpu.SemaphoreType.DMA((_NBUF,)),
    ],
)


def _ln_rows(y, g, bt):
    mu = jnp.mean(y, axis=-1, keepdims=True)
    var = jnp.mean((y - mu) ** 2, axis=-1, keepdims=True)
    return (y - mu) / jnp.sqrt(var + 1e-5) * g + bt


def _mm(a, b):
    return jnp.dot(a, b, preferred_element_type=jnp.float32)


def _enc_body(x_ref, w1, b1, w2, b2, g, bt, o_ref):
    t = jnp.maximum(_mm(x_ref[...], w1[...]) + b1[...], 0.0)
    y = _mm(t, w2[...]) + b2[...]
    o_ref[...] = _ln_rows(y, g[...], bt[...])


def _full(ref_shape):
    return pl.BlockSpec(ref_shape, lambda i: (0, 0))


def _encode(inp, w1, b1, w2, b2, g, bt, rb):
    n, d = inp.shape
    grid = n // rb
    return pl.pallas_call(
        _enc_body,
        grid=(grid,),
        in_specs=[
            pl.BlockSpec((rb, d), lambda i: (i, 0)),
            _full(w1.shape), _full(b1.shape), _full(w2.shape),
            _full(b2.shape), _full(g.shape), _full(bt.shape),
        ],
        out_specs=pl.BlockSpec((rb, H), lambda i: (i, 0)),
        out_shape=jax.ShapeDtypeStruct((n, H), jnp.float32),
    )(inp, w1, b1, w2, b2, g, bt)


def _upd_body(h_ref, a0_ref, a1_ref, A1, a1b, A2, a2b, g, bt, o_ref):
    h = h_ref[...]
    z = h + a0_ref[0] + a1_ref[0]
    t = jnp.maximum(_mm(z, A1[...]) + a1b[...], 0.0)
    hh = _mm(t, A2[...]) + a2b[...]
    o_ref[...] = _ln_rows(h + jnp.maximum(hh, 0.0), g[...], bt[...])


def _layer_update(h, parts, A1, a1b, A2, a2b, g, bt, rb=2000):
    grid = N // rb
    blk = pl.BlockSpec((rb, H), lambda i: (i, 0))
    p0 = pl.BlockSpec((1, rb, H), lambda i: (0, i, 0))
    p1 = pl.BlockSpec((1, rb, H), lambda i: (1, i, 0))
    return pl.pallas_call(
        _upd_body,
        grid=(grid,),
        in_specs=[blk, p0, p1, _full(A1.shape), _full(a1b.shape),
                  _full(A2.shape), _full(a2b.shape), _full(g.shape),
                  _full(bt.shape)],
        out_specs=blk,
        out_shape=jax.ShapeDtypeStruct((N, H), jnp.float32),
    )(h, parts, parts, A1, a1b, A2, a2b, g, bt)


def _heads_body(h_ref, dW1, db1, dW2, db2, sW1, sb1, sW2, sb2,
                feats, spW1, spb1, spW2, spb2, spg, spbt, spW3, spb3, lb,
                rawu_ref, u_ref, logs_ref, s_ref, safety_ref, disp_ref):
    h = h_ref[...]
    t = jnp.maximum(_mm(h, dW1[...]) + db1[...], 0.0)
    raw = _mm(t, dW2[...]) + db2[...]
    rms = jnp.maximum(jnp.sqrt(jnp.sum(raw * raw) / N), 1e-8)

    t2 = jnp.maximum(_mm(h, sW1[...]) + sb1[...], 0.0)
    ls = jnp.clip(_mm(t2, sW2[...]) + sb2[...], CLAMP_LO, CLAMP_HI)
    logs_ref[...] = ls
    sv = jnp.exp(ls)
    s_ref[...] = sv
    safety_ref[...] = YIELD / (sv + 1e-8)

    hs = jnp.maximum(_mm(feats[...], spW1[...]) + spb1[...], 0.0)
    hs = _mm(hs, spW2[...]) + spb2[...]
    hs = jnp.maximum(_ln_rows(hs, spg[...], spbt[...]), 0.0)
    lm = _mm(hs, spW3[...]) + spb3[...]
    lm = LOG_MULT_BOUND * jnp.tanh(lm / LOG_MULT_BOUND)
    base = MIN_DISP_SCALE + jnp.log1p(jnp.exp(lb[0, 0]))
    disp = jnp.maximum(base * jnp.exp(lm), MIN_DISP_SCALE)
    disp_ref[...] = disp

    ru = raw / rms
    rawu_ref[...] = ru
    u_ref[...] = ru * disp[0, 0]


def _heads(h, dW1, db1, dW2, db2, sW1, sb1, sW2, sb2,
           feats, spW1, spb1, spW2, spb2, spg, spbt, spW3, spb3, lb):
    return pl.pallas_call(
        _heads_body,
        out_shape=[
            jax.ShapeDtypeStruct((N, 3), jnp.float32),
            jax.ShapeDtypeStruct((N, 3), jnp.float32),
            jax.ShapeDtypeStruct((N, 1), jnp.float32),
            jax.ShapeDtypeStruct((N, 1), jnp.float32),
            jax.ShapeDtypeStruct((N, 1), jnp.float32),
            jax.ShapeDtypeStruct((1, 1), jnp.float32),
        ],
    )(h, dW1, db1, dW2, db2, sW1, sb1, sW2, sb2,
      feats, spW1, spb1, spW2, spb2, spg, spbt, spW3, spb3, lb)


def _row(v):
    return v.reshape(1, -1)


def kernel(x, edge_index, edge_attr, params):
    src = jnp.asarray(edge_index[0], jnp.int32)
    dst = jnp.asarray(edge_index[1], jnp.int32)

    pe = params['node_enc']
    h = _encode(x, pe['W1'], _row(pe['b1']), pe['W2'], _row(pe['b2']),
                _row(pe['g']), _row(pe['bt']), rb=2000)
    pg = params['edge_enc']
    ea = _encode(edge_attr, pg['W1'], _row(pg['b1']), pg['W2'], _row(pg['b2']),
                 _row(pg['g']), _row(pg['bt']), rb=2560)

    for cp in params['convs']:
        parts = _sc_aggregate(h, src, dst, ea)
        h = _layer_update(h, parts, cp['A1'], _row(cp['a1']), cp['A2'],
                          _row(cp['a2']), _row(cp['g']), _row(cp['bt']))

    one = jnp.ones((1, 1), dtype=jnp.float32)
    logF = jnp.log(one + 1.0)
    logE = jnp.log(jnp.full((1, 1), 2.1e11, dtype=jnp.float32) + 1e-12)
    nu = jnp.full((1, 1), 0.3, dtype=jnp.float32)
    logL = jnp.log(one + 1e-6)
    logI = jnp.log(one + 1e-18)
    phys = logF + 3.0 * logL - logE - logI
    feats = jnp.concatenate([logF, logE, nu, logL, logI, phys], axis=-1)

    dp = params['disp_head']
    st = params['stress_head']
    sp = params['scale_mlp']
    raw_u, u, log_s, s, safety, disp = _heads(
        h, dp['W1'], _row(dp['b1']), dp['W2'], _row(dp['b2']),
        st['W1'], _row(st['b1']), st['W2'], _row(st['b2']),
        feats, sp['W1'], _row(sp['b1']), sp['W2'], _row(sp['b2']),
        _row(sp['g']), _row(sp['bt']), sp['W3'], _row(sp['b3']),
        params['log_base'].reshape(1, 1))

    return (u, raw_u, s, log_s, disp[0, 0], disp, safety)


# speedup vs baseline: 2.1292x; 1.0485x over previous
import functools
import jax
import jax.numpy as jnp
from jax import lax
from jax.experimental import pallas as pl
from jax.experimental.pallas import tpu as pltpu
from jax.experimental.pallas import tpu_sc as plsc

N = 10000
E = 320000
H = 128
MIN_DISP_SCALE = 1e-06
LOG_MULT_BOUND = 4.0
CLAMP_LO, CLAMP_HI = 0.0, 30.0
YIELD = 2.5e8

_NC = 2
_NS = 16
_NW = _NC * _NS
_L = 16
_C = 80
_EW = E // _NW
_KW = _EW // _C
_NPAD = 10240
_ZR = 128
_RPT = _NPAD // _NS


_NBUF = 4


def _agg_body(h_hbm, src_hbm, dst_hbm, ea_hbm, out_hbm,
              agg_sh, sidx, didx, rows_v, sisem, disem, esem, gsem, ssem):
    c = lax.axis_index("c")
    s = lax.axis_index("s")
    wid = s * _NC + c

    zero16 = jnp.zeros((_L,), jnp.float32)

    def zrow(r, carry):
        for cc in range(H // _L):
            rows_v[0, r, pl.ds(cc * _L, _L)] = zero16
        return carry

    lax.fori_loop(0, _C, zrow, 0)
    base_rows = s * _RPT
    for j in range(_RPT // _C):
        pltpu.sync_copy(rows_v.at[0], agg_sh.at[pl.ds(base_rows + j * _C, _C)])
    plsc.subcore_barrier()

    ebase = wid * _EW

    def idx_issue(g, b):
        off = ebase + g * _C
        pltpu.async_copy(src_hbm.at[pl.ds(off, _C)], sidx.at[b], sisem.at[b])
        pltpu.async_copy(dst_hbm.at[pl.ds(off, _C)], didx.at[b], disem.at[b])

    def sidx_wait(b):
        pltpu.make_async_copy(src_hbm.at[pl.ds(0, _C)], sidx.at[b],
                              sisem.at[b]).wait()

    def didx_wait(b):
        pltpu.make_async_copy(dst_hbm.at[pl.ds(0, _C)], didx.at[b],
                              disem.at[b]).wait()

    def ea_issue(g, b):
        pltpu.async_copy(ea_hbm.at[pl.ds(ebase + g * _C, _C)],
                         rows_v.at[b], esem.at[b])

    def ea_wait(b):
        pltpu.make_async_copy(ea_hbm.at[pl.ds(0, _C)], rows_v.at[b],
                              esem.at[b]).wait()

    def gather_issue(b):
        pltpu.async_copy(h_hbm.at[sidx.at[b]], rows_v.at[b], gsem.at[b],
                         add=True)

    def gather_wait(b):
        pltpu.make_async_copy(h_hbm.at[pl.ds(0, _C)], rows_v.at[b],
                              gsem.at[b]).wait()

    def scatter_issue(b):
        pltpu.async_copy(rows_v.at[b], agg_sh.at[didx.at[b]], ssem.at[b],
                         add=True)

    def scatter_wait(b):
        pltpu.make_async_copy(rows_v.at[b], agg_sh.at[pl.ds(0, _C)],
                              ssem.at[b]).wait()

    def stage_a(g, b, first=False):
        if not first:
            scatter_wait(b)
        idx_issue(g, b)
        ea_issue(g, b)

    def stage_b(b):
        sidx_wait(b)
        ea_wait(b)
        gather_issue(b)

    def compute_scatter(b):
        gather_wait(b)

        def crow(r, carry):
            for cc in range(H // _L):
                sl = pl.ds(cc * _L, _L)
                rows_v[b, r, sl] = jnp.maximum(rows_v[b, r, sl], 0.0)
            return carry

        lax.fori_loop(0, _C, crow, 0)
        didx_wait(b)
        scatter_issue(b)

    stage_a(0, 0, first=True)
    stage_a(1, 1, first=True)
    stage_a(2, 2, first=True)
    stage_b(0)
    stage_b(1)
    stage_a(3, 3, first=True)
    stage_b(2)
    compute_scatter(0)

    def group(gg, carry):
        for j in range(_NBUF):
            g = 1 + gg * _NBUF + j
            b = (1 + j) % _NBUF
            stage_a(g + 3, (b + 3) % _NBUF)
            stage_b((b + 2) % _NBUF)
            compute_scatter(b)
        return carry

    lax.fori_loop(0, (_KW - 5) // _NBUF, group, 0)

    stage_a(_KW - 1, 0)
    stage_b(3)
    compute_scatter(1)
    stage_b(0)
    compute_scatter(2)
    compute_scatter(3)
    compute_scatter(0)
    for b in range(_NBUF):
        scatter_wait(b)

    plsc.subcore_barrier()

    for j in range(_RPT // _C):
        r0 = base_rows + j * _C
        pltpu.sync_copy(agg_sh.at[pl.ds(r0, _C)], out_hbm.at[c, pl.ds(r0, _C)])


_sc_aggregate = pl.kernel(
    _agg_body,
    out_type=jax.ShapeDtypeStruct((_NC, _NPAD, H), jnp.float32),
    mesh=plsc.VectorSubcoreMesh(core_axis_name="c", subcore_axis_name="s"),
    scratch_types=[
        pltpu.VMEM_SHARED((_NPAD, H), jnp.float32),
        pltpu.VMEM((_NBUF, _C), jnp.int32),
        pltpu.VMEM((_NBUF, _C), jnp.int32),
        pltpu.VMEM((_NBUF, _C, H), jnp.float32),
        pltpu.SemaphoreType.DMA((_NBUF,)),
        pltpu.SemaphoreType.DMA((_NBUF,)),
        pltpu.SemaphoreType.DMA((_NBUF,)),
        pltpu.SemaphoreType.DMA((_NBUF,)),
        pltpu.SemaphoreType.DMA((_NBUF,)),
    ],
)


def _ln_rows(y, g, bt):
    mu = jnp.mean(y, axis=-1, keepdims=True)
    var = jnp.mean((y - mu) ** 2, axis=-1, keepdims=True)
    inv = 1.0 / jnp.sqrt(var + 1e-5)
    return (y - mu) * inv * g + bt


def _mm(a, b):
    return jnp.dot(a, b, preferred_element_type=jnp.float32)


def _enc_body(x_ref, w1, b1, w2, b2, g, bt, o_ref):
    t = jnp.maximum(_mm(x_ref[...], w1[...]) + b1[...], 0.0)
    y = _mm(t, w2[...]) + b2[...]
    o_ref[...] = _ln_rows(y, g[...], bt[...])


def _full(ref_shape):
    return pl.BlockSpec(ref_shape, lambda i: (0, 0))


def _encode(inp, w1, b1, w2, b2, g, bt, rb):
    n, d = inp.shape
    grid = n // rb
    return pl.pallas_call(
        _enc_body,
        grid=(grid,),
        in_specs=[
            pl.BlockSpec((rb, d), lambda i: (i, 0)),
            _full(w1.shape), _full(b1.shape), _full(w2.shape),
            _full(b2.shape), _full(g.shape), _full(bt.shape),
        ],
        out_specs=pl.BlockSpec((rb, H), lambda i: (i, 0)),
        out_shape=jax.ShapeDtypeStruct((n, H), jnp.float32),
    )(inp, w1, b1, w2, b2, g, bt)


def _upd_body(h_ref, a0_ref, a1_ref, A1, a1b, A2, a2b, g, bt, o_ref):
    h = h_ref[...]
    z = h + a0_ref[0] + a1_ref[0]
    t = jnp.maximum(_mm(z, A1[...]) + a1b[...], 0.0)
    hh = _mm(t, A2[...]) + a2b[...]
    o_ref[...] = _ln_rows(h + jnp.maximum(hh, 0.0), g[...], bt[...])


def _layer_update(h, parts, A1, a1b, A2, a2b, g, bt, rb=2000):
    grid = N // rb
    blk = pl.BlockSpec((rb, H), lambda i: (i, 0))
    p0 = pl.BlockSpec((1, rb, H), lambda i: (0, i, 0))
    p1 = pl.BlockSpec((1, rb, H), lambda i: (1, i, 0))
    return pl.pallas_call(
        _upd_body,
        grid=(grid,),
        in_specs=[blk, p0, p1, _full(A1.shape), _full(a1b.shape),
                  _full(A2.shape), _full(a2b.shape), _full(g.shape),
                  _full(bt.shape)],
        out_specs=blk,
        out_shape=jax.ShapeDtypeStruct((N, H), jnp.float32),
    )(h, parts, parts, A1, a1b, A2, a2b, g, bt)


def _heads_body(h_ref, dW1, db1, dW2, db2, sW1, sb1, sW2, sb2,
                feats, spW1, spb1, spW2, spb2, spg, spbt, spW3, spb3, lb,
                rawu_ref, u_ref, logs_ref, s_ref, safety_ref, disp_ref):
    h = h_ref[...]
    t = jnp.maximum(_mm(h, dW1[...]) + db1[...], 0.0)
    raw = _mm(t, dW2[...]) + db2[...]
    rms = jnp.maximum(jnp.sqrt(jnp.sum(raw * raw) / N), 1e-8)

    t2 = jnp.maximum(_mm(h, sW1[...]) + sb1[...], 0.0)
    ls = jnp.clip(_mm(t2, sW2[...]) + sb2[...], CLAMP_LO, CLAMP_HI)
    logs_ref[...] = ls
    sv = jnp.exp(ls)
    s_ref[...] = sv
    safety_ref[...] = YIELD / (sv + 1e-8)

    hs = jnp.maximum(_mm(feats[...], spW1[...]) + spb1[...], 0.0)
    hs = _mm(hs, spW2[...]) + spb2[...]
    hs = jnp.maximum(_ln_rows(hs, spg[...], spbt[...]), 0.0)
    lm = _mm(hs, spW3[...]) + spb3[...]
    lm = LOG_MULT_BOUND * jnp.tanh(lm / LOG_MULT_BOUND)
    base = MIN_DISP_SCALE + jnp.log1p(jnp.exp(lb[0, 0]))
    disp = jnp.maximum(base * jnp.exp(lm), MIN_DISP_SCALE)
    disp_ref[...] = disp

    ru = raw / rms
    rawu_ref[...] = ru
    u_ref[...] = ru * disp[0, 0]


def _heads(h, dW1, db1, dW2, db2, sW1, sb1, sW2, sb2,
           feats, spW1, spb1, spW2, spb2, spg, spbt, spW3, spb3, lb):
    return pl.pallas_call(
        _heads_body,
        out_shape=[
            jax.ShapeDtypeStruct((N, 3), jnp.float32),
            jax.ShapeDtypeStruct((N, 3), jnp.float32),
            jax.ShapeDtypeStruct((N, 1), jnp.float32),
            jax.ShapeDtypeStruct((N, 1), jnp.float32),
            jax.ShapeDtypeStruct((N, 1), jnp.float32),
            jax.ShapeDtypeStruct((1, 1), jnp.float32),
        ],
    )(h, dW1, db1, dW2, db2, sW1, sb1, sW2, sb2,
      feats, spW1, spb1, spW2, spb2, spg, spbt, spW3, spb3, lb)


def _row(v):
    return v.reshape(1, -1)


def kernel(x, edge_index, edge_attr, params):
    src = jnp.asarray(edge_index[0], jnp.int32)
    dst = jnp.asarray(edge_index[1], jnp.int32)

    pe = params['node_enc']
    h = _encode(x, pe['W1'], _row(pe['b1']), pe['W2'], _row(pe['b2']),
                _row(pe['g']), _row(pe['bt']), rb=2000)
    pg = params['edge_enc']
    ea = _encode(edge_attr, pg['W1'], _row(pg['b1']), pg['W2'], _row(pg['b2']),
                 _row(pg['g']), _row(pg['bt']), rb=6400)

    for cp in params['convs']:
        parts = _sc_aggregate(h, src, dst, ea)
        h = _layer_update(h, parts, cp['A1'], _row(cp['a1']), cp['A2'],
                          _row(cp['a2']), _row(cp['g']), _row(cp['bt']))

    one = jnp.ones((1, 1), dtype=jnp.float32)
    logF = jnp.log(one + 1.0)
    logE = jnp.log(jnp.full((1, 1), 2.1e11, dtype=jnp.float32) + 1e-12)
    nu = jnp.full((1, 1), 0.3, dtype=jnp.float32)
    logL = jnp.log(one + 1e-6)
    logI = jnp.log(one + 1e-18)
    phys = logF + 3.0 * logL - logE - logI
    feats = jnp.concatenate([logF, logE, nu, logL, logI, phys], axis=-1)

    dp = params['disp_head']
    st = params['stress_head']
    sp = params['scale_mlp']
    raw_u, u, log_s, s, safety, disp = _heads(
        h, dp['W1'], _row(dp['b1']), dp['W2'], _row(dp['b2']),
        st['W1'], _row(st['b1']), st['W2'], _row(st['b2']),
        feats, sp['W1'], _row(sp['b1']), sp['W2'], _row(sp['b2']),
        _row(sp['g']), _row(sp['bt']), sp['W3'], _row(sp['b3']),
        params['log_base'].reshape(1, 1))

    return (u, raw_u, s, log_s, disp[0, 0], disp, safety)


# speedup vs baseline: 2.1303x; 1.0005x over previous
import jax
import jax.numpy as jnp
from jax import lax
from jax.experimental import pallas as pl
from jax.experimental.pallas import tpu as pltpu
from jax.experimental.pallas import tpu_sc as plsc

N = 10000
E = 320000
H = 128
MIN_DISP_SCALE = 1e-06
LOG_MULT_BOUND = 4.0
CLAMP_LO, CLAMP_HI = 0.0, 30.0
YIELD = 2.5e8

_NC = 2
_NS = 16
_NW = _NC * _NS
_L = 16
_C = 80
_EW = E // _NW
_KW = _EW // _C
_NPAD = 10240
_RPT = _NPAD // _NS
_NBUF = 4


def _agg_body(h_hbm, src_hbm, dst_hbm, ea_hbm, out_hbm,
              agg_sh, sidx, didx, rows_v, sisem, disem, esem, gsem, ssem):
    c = lax.axis_index("c")
    s = lax.axis_index("s")
    wid = s * _NC + c

    zero16 = jnp.zeros((_L,), jnp.float32)

    def zrow(r, carry):
        for cc in range(H // _L):
            rows_v[0, r, pl.ds(cc * _L, _L)] = zero16
        return carry

    lax.fori_loop(0, _C, zrow, 0)
    base_rows = s * _RPT
    for j in range(_RPT // _C):
        pltpu.sync_copy(rows_v.at[0], agg_sh.at[pl.ds(base_rows + j * _C, _C)])
    plsc.subcore_barrier()

    ebase = wid * _EW

    def idx_issue(g, b):
        off = ebase + g * _C
        pltpu.async_copy(src_hbm.at[pl.ds(off, _C)], sidx.at[b], sisem.at[b])
        pltpu.async_copy(dst_hbm.at[pl.ds(off, _C)], didx.at[b], disem.at[b])

    def sidx_wait(b):
        pltpu.make_async_copy(src_hbm.at[pl.ds(0, _C)], sidx.at[b],
                              sisem.at[b]).wait()

    def didx_wait(b):
        pltpu.make_async_copy(dst_hbm.at[pl.ds(0, _C)], didx.at[b],
                              disem.at[b]).wait()

    def ea_issue(g, b):
        pltpu.async_copy(ea_hbm.at[pl.ds(ebase + g * _C, _C)],
                         rows_v.at[b], esem.at[b])

    def ea_wait(b):
        pltpu.make_async_copy(ea_hbm.at[pl.ds(0, _C)], rows_v.at[b],
                              esem.at[b]).wait()

    def gather_issue(b):
        pltpu.async_copy(h_hbm.at[sidx.at[b]], rows_v.at[b], gsem.at[b],
                         add=True)

    def gather_wait(b):
        pltpu.make_async_copy(h_hbm.at[pl.ds(0, _C)], rows_v.at[b],
                              gsem.at[b]).wait()

    def scatter_issue(b):
        pltpu.async_copy(rows_v.at[b], agg_sh.at[didx.at[b]], ssem.at[b],
                         add=True)

    def scatter_wait(b):
        pltpu.make_async_copy(rows_v.at[b], agg_sh.at[pl.ds(0, _C)],
                              ssem.at[b]).wait()

    def stage_a(g, b, first=False):
        if not first:
            scatter_wait(b)
        idx_issue(g, b)
        ea_issue(g, b)

    def stage_b(b):
        sidx_wait(b)
        ea_wait(b)
        gather_issue(b)

    def compute_scatter(b):
        gather_wait(b)

        def crow(r, carry):
            for cc in range(H // _L):
                sl = pl.ds(cc * _L, _L)
                rows_v[b, r, sl] = jnp.maximum(rows_v[b, r, sl], 0.0)
            return carry

        lax.fori_loop(0, _C, crow, 0)
        didx_wait(b)
        scatter_issue(b)

    stage_a(0, 0, first=True)
    stage_a(1, 1, first=True)
    stage_a(2, 2, first=True)
    stage_b(0)
    stage_b(1)
    stage_a(3, 3, first=True)
    stage_b(2)
    compute_scatter(0)

    def group(gg, carry):
        for j in range(_NBUF):
            g = 1 + gg * _NBUF + j
            b = (1 + j) % _NBUF
            stage_a(g + 3, (b + 3) % _NBUF)
            stage_b((b + 2) % _NBUF)
            compute_scatter(b)
        return carry

    lax.fori_loop(0, (_KW - 5) // _NBUF, group, 0)

    stage_a(_KW - 1, 0)
    stage_b(3)
    compute_scatter(1)
    stage_b(0)
    compute_scatter(2)
    compute_scatter(3)
    compute_scatter(0)
    for b in range(_NBUF):
        scatter_wait(b)

    plsc.subcore_barrier()

    for j in range(_RPT // _C):
        r0 = base_rows + j * _C
        pltpu.sync_copy(agg_sh.at[pl.ds(r0, _C)], out_hbm.at[c, pl.ds(r0, _C)])


_sc_aggregate = pl.kernel(
    _agg_body,
    out_type=jax.ShapeDtypeStruct((_NC, _NPAD, H), jnp.float32),
    mesh=plsc.VectorSubcoreMesh(core_axis_name="c", subcore_axis_name="s"),
    scratch_types=[
        pltpu.VMEM_SHARED((_NPAD, H), jnp.float32),
        pltpu.VMEM((_NBUF, _C), jnp.int32),
        pltpu.VMEM((_NBUF, _C), jnp.int32),
        pltpu.VMEM((_NBUF, _C, H), jnp.float32),
        pltpu.SemaphoreType.DMA((_NBUF,)),
        pltpu.SemaphoreType.DMA((_NBUF,)),
        pltpu.SemaphoreType.DMA((_NBUF,)),
        pltpu.SemaphoreType.DMA((_NBUF,)),
        pltpu.SemaphoreType.DMA((_NBUF,)),
    ],
)


def _ln_rows(y, g, bt):
    mu = jnp.mean(y, axis=-1, keepdims=True)
    var = jnp.mean((y - mu) ** 2, axis=-1, keepdims=True)
    inv = 1.0 / jnp.sqrt(var + 1e-5)
    return (y - mu) * inv * g + bt


def _mm(a, b):
    return jnp.dot(a, b, preferred_element_type=jnp.float32)


def _enc_body(x_ref, w1, b1, w2, b2, g, bt, o_ref):
    t = jnp.maximum(_mm(x_ref[...], w1[...]) + b1[...], 0.0)
    y = _mm(t, w2[...]) + b2[...]
    o_ref[...] = _ln_rows(y, g[...], bt[...])


def _full(ref_shape):
    return pl.BlockSpec(ref_shape, lambda i: (0, 0))


def _encode(inp, w1, b1, w2, b2, g, bt, rb):
    n, d = inp.shape
    grid = n // rb
    return pl.pallas_call(
        _enc_body,
        grid=(grid,),
        in_specs=[
            pl.BlockSpec((rb, d), lambda i: (i, 0)),
            _full(w1.shape), _full(b1.shape), _full(w2.shape),
            _full(b2.shape), _full(g.shape), _full(bt.shape),
        ],
        out_specs=pl.BlockSpec((rb, H), lambda i: (i, 0)),
        out_shape=jax.ShapeDtypeStruct((n, H), jnp.float32),
    )(inp, w1, b1, w2, b2, g, bt)


def _upd_body(h_ref, a0_ref, a1_ref, A1, a1b, A2, a2b, g, bt, o_ref):
    h = h_ref[...]
    z = h + a0_ref[0] + a1_ref[0]
    t = jnp.maximum(_mm(z, A1[...]) + a1b[...], 0.0)
    hh = _mm(t, A2[...]) + a2b[...]
    o_ref[...] = _ln_rows(h + jnp.maximum(hh, 0.0), g[...], bt[...])


def _layer_update(h, parts, A1, a1b, A2, a2b, g, bt, rb=2000):
    grid = N // rb
    blk = pl.BlockSpec((rb, H), lambda i: (i, 0))
    p0 = pl.BlockSpec((1, rb, H), lambda i: (0, i, 0))
    p1 = pl.BlockSpec((1, rb, H), lambda i: (1, i, 0))
    return pl.pallas_call(
        _upd_body,
        grid=(grid,),
        in_specs=[blk, p0, p1, _full(A1.shape), _full(a1b.shape),
                  _full(A2.shape), _full(a2b.shape), _full(g.shape),
                  _full(bt.shape)],
        out_specs=blk,
        out_shape=jax.ShapeDtypeStruct((N, H), jnp.float32),
    )(h, parts, parts, A1, a1b, A2, a2b, g, bt)


def _heads_body(h_ref, dW1, db1, dW2, db2, sW1, sb1, sW2, sb2,
                feats, spW1, spb1, spW2, spb2, spg, spbt, spW3, spb3, lb,
                rawu_ref, u_ref, logs_ref, s_ref, safety_ref, disp_ref):
    h = h_ref[...]
    t = jnp.maximum(_mm(h, dW1[...]) + db1[...], 0.0)
    raw = _mm(t, dW2[...]) + db2[...]
    rms = jnp.maximum(jnp.sqrt(jnp.sum(raw * raw) / N), 1e-8)

    t2 = jnp.maximum(_mm(h, sW1[...]) + sb1[...], 0.0)
    ls = jnp.clip(_mm(t2, sW2[...]) + sb2[...], CLAMP_LO, CLAMP_HI)
    logs_ref[...] = ls
    sv = jnp.exp(ls)
    s_ref[...] = sv
    safety_ref[...] = YIELD / (sv + 1e-8)

    hs = jnp.maximum(_mm(feats[...], spW1[...]) + spb1[...], 0.0)
    hs = _mm(hs, spW2[...]) + spb2[...]
    hs = jnp.maximum(_ln_rows(hs, spg[...], spbt[...]), 0.0)
    lm = _mm(hs, spW3[...]) + spb3[...]
    lm = LOG_MULT_BOUND * jnp.tanh(lm / LOG_MULT_BOUND)
    lbv = lb[0, 0]
    softplus = jnp.maximum(lbv, 0.0) + jnp.log1p(jnp.exp(-jnp.abs(lbv)))
    base = MIN_DISP_SCALE + softplus
    disp = jnp.maximum(base * jnp.exp(lm), MIN_DISP_SCALE)
    disp_ref[...] = disp

    ru = raw / rms
    rawu_ref[...] = ru
    u_ref[...] = ru * disp[0, 0]


def _heads(h, dW1, db1, dW2, db2, sW1, sb1, sW2, sb2,
           feats, spW1, spb1, spW2, spb2, spg, spbt, spW3, spb3, lb):
    return pl.pallas_call(
        _heads_body,
        out_shape=[
            jax.ShapeDtypeStruct((N, 3), jnp.float32),
            jax.ShapeDtypeStruct((N, 3), jnp.float32),
            jax.ShapeDtypeStruct((N, 1), jnp.float32),
            jax.ShapeDtypeStruct((N, 1), jnp.float32),
            jax.ShapeDtypeStruct((N, 1), jnp.float32),
            jax.ShapeDtypeStruct((1, 1), jnp.float32),
        ],
    )(h, dW1, db1, dW2, db2, sW1, sb1, sW2, sb2,
      feats, spW1, spb1, spW2, spb2, spg, spbt, spW3, spb3, lb)


def _row(v):
    return v.reshape(1, -1)


def kernel(x, edge_index, edge_attr, params):
    src = jnp.asarray(edge_index[0], jnp.int32)
    dst = jnp.asarray(edge_index[1], jnp.int32)

    pe = params['node_enc']
    h = _encode(x, pe['W1'], _row(pe['b1']), pe['W2'], _row(pe['b2']),
                _row(pe['g']), _row(pe['bt']), rb=2000)
    pg = params['edge_enc']
    ea = _encode(edge_attr, pg['W1'], _row(pg['b1']), pg['W2'], _row(pg['b2']),
                 _row(pg['g']), _row(pg['bt']), rb=6400)

    for cp in params['convs']:
        parts = _sc_aggregate(h, src, dst, ea)
        h = _layer_update(h, parts, cp['A1'], _row(cp['a1']), cp['A2'],
                          _row(cp['a2']), _row(cp['g']), _row(cp['bt']))

    one = jnp.ones((1, 1), dtype=jnp.float32)
    logF = jnp.log(one + 1.0)
    logE = jnp.log(jnp.full((1, 1), 2.1e11, dtype=jnp.float32) + 1e-12)
    nu = jnp.full((1, 1), 0.3, dtype=jnp.float32)
    logL = jnp.log(one + 1e-6)
    logI = jnp.log(one + 1e-18)
    phys = logF + 3.0 * logL - logE - logI
    feats = jnp.concatenate([logF, logE, nu, logL, logI, phys], axis=-1)

    dp = params['disp_head']
    st = params['stress_head']
    sp = params['scale_mlp']
    raw_u, u, log_s, s, safety, disp = _heads(
        h, dp['W1'], _row(dp['b1']), dp['W2'], _row(dp['b2']),
        st['W1'], _row(st['b1']), st['W2'], _row(st['b2']),
        feats, sp['W1'], _row(sp['b1']), sp['W2'], _row(sp['b2']),
        _row(sp['g']), _row(sp['bt']), sp['W3'], _row(sp['b3']),
        params['log_base'].reshape(1, 1))

    return (u, raw_u, s, log_s, disp[0, 0], disp, safety)
